# Initial kernel scaffold; baseline (speedup 1.0000x reference)
import jax
import jax.numpy as jnp
from jax.experimental import pallas as pl


def kernel(x, edge_index, edge_attr, W_in, b_in, ln_g, ln_b, emb, W1, att1, n1_g, n1_b, W2, att2, n2_g, n2_b, Ws1, bs1, Ws2, bs2):
    raise NotImplementedError("write your pallas kernel here")


# speedup vs baseline: 9.9083x; 9.9083x over previous
import functools

import jax
import jax.numpy as jnp
from jax import lax
from jax.experimental import pallas as pl
from jax.experimental.pallas import tpu as pltpu
from jax.experimental.pallas import tpu_sc as plsc

N = 1024
E = 16384
FEAT = 256
HID = 128
NS = 16
EPT = E // NS
NPT = N // NS
NV = EPT // 16
EPC = EPT // 2
NRC = EPC // 128


def _ln(t, g, b):
    m = jnp.mean(t, axis=-1, keepdims=True)
    v = jnp.mean((t - m) ** 2, axis=-1, keepdims=True)
    return (t - m) * lax.rsqrt(v + 1e-5) * g + b


def _gelu(t):
    return 0.5 * t * (1.0 + lax.erf(t / jnp.sqrt(2.0).astype(jnp.float32)))


def _elu(t):
    return jnp.where(t > 0, t, jnp.exp(t) - 1.0)


def _tc1_body(x_ref, wint_ref, b_ref, lg_ref, lb_ref, w1t_ref, as_ref, ad_ref,
              h_ref, hw_ref, s1_ref, s2_ref):
    t = jnp.dot(x_ref[...], wint_ref[...], preferred_element_type=jnp.float32)
    h = _gelu(_ln(t + b_ref[...], lg_ref[...], lb_ref[...]))
    h_ref[...] = h
    hw = jnp.dot(h, w1t_ref[...], preferred_element_type=jnp.float32)
    hw_ref[...] = hw
    s1_ref[...] = jnp.sum(hw * as_ref[...], axis=-1, keepdims=True)
    s2_ref[...] = jnp.sum(hw * ad_ref[...], axis=-1, keepdims=True)


def _tc1(x, wint, b, lg, lb, w1t, a_s, a_d):
    return pl.pallas_call(
        _tc1_body,
        out_shape=[
            jax.ShapeDtypeStruct((N, HID), jnp.float32),
            jax.ShapeDtypeStruct((N, HID), jnp.float32),
            jax.ShapeDtypeStruct((N, 1), jnp.float32),
            jax.ShapeDtypeStruct((N, 1), jnp.float32),
        ],
    )(x, wint, b, lg, lb, w1t, a_s, a_d)


def _tc2_body(h_ref, agg_ref, ng_ref, nb_ref, wt_ref, as_ref, ad_ref,
              hn_ref, hw_ref, s1_ref, s2_ref):
    agg = agg_ref[0] + agg_ref[1]
    hn = _ln(h_ref[...] + _elu(agg), ng_ref[...], nb_ref[...])
    hn_ref[...] = hn
    hw = jnp.dot(hn, wt_ref[...], preferred_element_type=jnp.float32)
    hw_ref[...] = hw
    s1_ref[...] = jnp.sum(hw * as_ref[...], axis=-1, keepdims=True)
    s2_ref[...] = jnp.sum(hw * ad_ref[...], axis=-1, keepdims=True)


def _tc2(h, agg, ng, nb, wt, a_s, a_d):
    return pl.pallas_call(
        _tc2_body,
        out_shape=[
            jax.ShapeDtypeStruct((N, HID), jnp.float32),
            jax.ShapeDtypeStruct((N, HID), jnp.float32),
            jax.ShapeDtypeStruct((N, 1), jnp.float32),
            jax.ShapeDtypeStruct((N, 1), jnp.float32),
        ],
    )(h, agg, ng, nb, wt, a_s, a_d)


def _tc3_body(h_ref, agg_ref, ng_ref, nb_ref, ws1t_ref, bs1_ref, ws2t_ref,
              bs2_ref, out_ref):
    agg = agg_ref[0] + agg_ref[1]
    h2 = _ln(h_ref[...] + _elu(agg), ng_ref[...], nb_ref[...])
    g = jnp.mean(h2, axis=0, keepdims=True)
    hid = _gelu(jnp.dot(g, ws1t_ref[...], preferred_element_type=jnp.float32)
                + bs1_ref[...])
    out_ref[...] = (jnp.dot(hid, ws2t_ref[...],
                            preferred_element_type=jnp.float32) + bs2_ref[...])


def _tc3(h, agg, ng, nb, ws1t, bs1, ws2t, bs2):
    return pl.pallas_call(
        _tc3_body,
        out_shape=jax.ShapeDtypeStruct((1, 1), jnp.float32),
    )(h, agg, ng, nb, ws1t, bs1, ws2t, bs2)


def _sc_gat_body(src_hbm, dst_hbm, s1_hbm, s2_hbm, hw_hbm, out_hbm,
                 src_v, dst_v, gidx_v, sidx_v, s1_v, s2_v, a_v, m_v, red_v,
                 denp_v, comb_v, dsl_v, den_v, rows_v, out_v,
                 red_sh, parts_sh, den_sh, agg_sh):
    cid = lax.axis_index("c")
    sid = lax.axis_index("s")
    nbase = sid * NPT

    pltpu.sync_copy(src_hbm.at[sid], src_v)
    pltpu.sync_copy(dst_hbm.at[sid], dst_v)
    pltpu.sync_copy(s1_hbm, s1_v)
    pltpu.sync_copy(s2_hbm, s2_v)

    zf = jnp.zeros((16,), jnp.float32)
    for k in range(N // 16):
        denp_v[pl.ds(16 * k, 16)] = zf
    for r in range(NPT):
        for q in range(HID // 16):
            out_v[r, pl.ds(16 * q, 16)] = zf
    pltpu.sync_copy(out_v, agg_sh.at[pl.ds(nbase, NPT)])

    m = jnp.zeros((16,), jnp.float32)
    for k in range(NV):
        i1 = src_v[k, pl.ds(0, 16)]
        i2 = dst_v[k, pl.ds(0, 16)]
        g1 = plsc.load_gather(s1_v, [i1])
        g2 = plsc.load_gather(s2_v, [i2])
        a = g1 + g2
        a = jnp.where(a > 0, a, 0.2 * a)
        a_v[pl.ds(16 * k, 16)] = a
        m = jnp.maximum(m, a)
    m_v[...] = m
    pltpu.sync_copy(m_v, red_sh.at[sid])
    plsc.subcore_barrier()

    pltpu.sync_copy(red_sh, red_v)
    mm = red_v[0, pl.ds(0, 16)]
    for r in range(1, NS):
        mm = jnp.maximum(mm, red_v[r, pl.ds(0, 16)])
    c = jnp.max(mm)
    cvec = jnp.full((16,), c, jnp.float32)
    emc = jnp.exp(-cvec)

    for k in range(NV):
        a = a_v[pl.ds(16 * k, 16)]
        ea = jnp.exp(a - cvec)
        a_v[pl.ds(16 * k, 16)] = ea
        i2 = dst_v[k, pl.ds(0, 16)]
        plsc.addupdate_scatter(denp_v, [i2], ea - emc)
    pltpu.sync_copy(denp_v, parts_sh.at[pl.ds(sid * N, N)])
    plsc.subcore_barrier()

    for w in range(NS):
        pltpu.sync_copy(parts_sh.at[pl.ds(w * N + nbase, NPT)],
                        comb_v.at[pl.ds(w * NPT, NPT)])
    base = jnp.full((16,), float(E), jnp.float32) * emc
    for q in range(NPT // 16):
        acc = base
        for w in range(NS):
            acc = acc + comb_v[pl.ds(w * NPT + 16 * q, 16)]
        dsl_v[pl.ds(16 * q, 16)] = acc
    pltpu.sync_copy(dsl_v, den_sh.at[pl.ds(nbase, NPT)])
    plsc.subcore_barrier()

    pltpu.sync_copy(den_sh, den_v)
    for k in range(NV):
        i2 = dst_v[k, pl.ds(0, 16)]
        d = plsc.load_gather(den_v, [i2])
        a_v[pl.ds(16 * k, 16)] = a_v[pl.ds(16 * k, 16)] / d

    for j in range(NRC):
        for q in range(8):
            kk = cid * (NV // 2) + j * 8 + q
            gidx_v[j, pl.ds(16 * q, 16)] = src_v[kk, pl.ds(0, 16)]
            sidx_v[j, pl.ds(16 * q, 16)] = dst_v[kk, pl.ds(0, 16)]
    for j in range(NRC):
        pltpu.sync_copy(hw_hbm.at[gidx_v.at[j]],
                        rows_v.at[pl.ds(128 * j, 128)])

    ebase = cid * EPC

    def scale_body(e, carry):
        cf = plsc.load_gather(a_v, [jnp.full((16,), ebase, jnp.int32)
                                    + jnp.full((16,), e, jnp.int32)])
        for q in range(HID // 16):
            rows_v[e, pl.ds(16 * q, 16)] = rows_v[e, pl.ds(16 * q, 16)] * cf
        return carry

    lax.fori_loop(0, EPC, scale_body, 0)

    for j in range(NRC):
        pltpu.sync_copy(rows_v.at[pl.ds(128 * j, 128)],
                        agg_sh.at[sidx_v.at[j]], add=True)
    plsc.subcore_barrier()

    pltpu.sync_copy(agg_sh.at[pl.ds(nbase, NPT)], out_v)
    pltpu.sync_copy(out_v, out_hbm.at[cid, pl.ds(nbase, NPT)])


_SC_SCRATCH = [
    pltpu.VMEM((NV, 16), jnp.int32),
    pltpu.VMEM((NV, 16), jnp.int32),
    pltpu.VMEM((NRC, 128), jnp.int32),
    pltpu.VMEM((NRC, 128), jnp.int32),
    pltpu.VMEM((N,), jnp.float32),
    pltpu.VMEM((N,), jnp.float32),
    pltpu.VMEM((EPT,), jnp.float32),
    pltpu.VMEM((16,), jnp.float32),
    pltpu.VMEM((NS, 16), jnp.float32),
    pltpu.VMEM((N,), jnp.float32),
    pltpu.VMEM((NS * NPT,), jnp.float32),
    pltpu.VMEM((NPT,), jnp.float32),
    pltpu.VMEM((N,), jnp.float32),
    pltpu.VMEM((EPC, HID), jnp.float32),
    pltpu.VMEM((NPT, HID), jnp.float32),
    pltpu.VMEM_SHARED((NS, 16), jnp.float32),
    pltpu.VMEM_SHARED((NS * N,), jnp.float32),
    pltpu.VMEM_SHARED((N,), jnp.float32),
    pltpu.VMEM_SHARED((N, HID), jnp.float32),
]


@functools.lru_cache(maxsize=1)
def _build_sc_gat():
    return functools.partial(
        pl.kernel,
        out_type=jax.ShapeDtypeStruct((2, N, HID), jnp.float32),
        mesh=plsc.VectorSubcoreMesh(core_axis_name="c", subcore_axis_name="s"),
        scratch_types=_SC_SCRATCH,
        compiler_params=pltpu.CompilerParams(needs_layout_passes=False),
    )(_sc_gat_body)


def _sc_gat(src3, dst3, s1, s2, hw):
    return _build_sc_gat()(src3, dst3, s1, s2, hw)


def kernel(x, edge_index, edge_attr, W_in, b_in, ln_g, ln_b, emb, W1, att1,
           n1_g, n1_b, W2, att2, n2_g, n2_b, Ws1, bs1, Ws2, bs2):
    src3 = edge_index[0].reshape(NS, NV, 16)
    dst3 = edge_index[1].reshape(NS, NV, 16)
    r1 = lambda v: v.reshape(1, -1)

    h, hw1, s1a, s2a = _tc1(x, W_in.T, r1(b_in), r1(ln_g), r1(ln_b), W1.T,
                            att1[:, :HID], att1[:, HID:])
    agg1 = _sc_gat(src3, dst3, s1a.reshape(N), s2a.reshape(N), hw1)
    h1, hw2, s1b, s2b = _tc2(h, agg1, r1(n1_g), r1(n1_b), W2.T,
                             att2[:, :HID], att2[:, HID:])
    agg2 = _sc_gat(src3, dst3, s1b.reshape(N), s2b.reshape(N), hw2)
    return _tc3(h1, agg2, r1(n2_g), r1(n2_b), Ws1.T, r1(bs1), Ws2.T,
                bs2.reshape(1, 1))


# speedup vs baseline: 11.1800x; 1.1284x over previous
import functools

import jax
import jax.numpy as jnp
from jax import lax
from jax.experimental import pallas as pl
from jax.experimental.pallas import tpu as pltpu
from jax.experimental.pallas import tpu_sc as plsc

N = 1024
E = 16384
FEAT = 256
HID = 128
NS = 16
EPT = E // NS
NPT = N // NS
NV = EPT // 16
EPC = EPT // 2
NRC = EPC // 128


def _ln(t, g, b):
    m = jnp.mean(t, axis=-1, keepdims=True)
    v = jnp.mean((t - m) ** 2, axis=-1, keepdims=True)
    return (t - m) * lax.rsqrt(v + 1e-5) * g + b


def _gelu(t):
    return 0.5 * t * (1.0 + lax.erf(t / jnp.sqrt(2.0).astype(jnp.float32)))


def _elu(t):
    return jnp.where(t > 0, t, jnp.exp(t) - 1.0)


def _scores(hw, a_s, a_d, s1_ref, s2_ref, cb_ref):
    s1 = jnp.sum(hw * a_s, axis=-1, keepdims=True)
    s2 = jnp.sum(hw * a_d, axis=-1, keepdims=True)
    s1_ref[...] = s1
    s2_ref[...] = s2
    cb = jnp.maximum(0.0, jnp.max(s1) + jnp.max(s2))
    cb_ref[...] = jnp.zeros((1, HID), jnp.float32) + cb


def _tc1_body(x_ref, wint_ref, b_ref, lg_ref, lb_ref, w1t_ref, as_ref, ad_ref,
              h_ref, hw_ref, s1_ref, s2_ref, cb_ref):
    t = jnp.dot(x_ref[...], wint_ref[...], preferred_element_type=jnp.float32)
    h = _gelu(_ln(t + b_ref[...], lg_ref[...], lb_ref[...]))
    h_ref[...] = h
    hw = jnp.dot(h, w1t_ref[...], preferred_element_type=jnp.float32)
    hw_ref[...] = hw
    _scores(hw, as_ref[...], ad_ref[...], s1_ref, s2_ref, cb_ref)


def _tc1(x, wint, b, lg, lb, w1t, a_s, a_d):
    return pl.pallas_call(
        _tc1_body,
        out_shape=[
            jax.ShapeDtypeStruct((N, HID), jnp.float32),
            jax.ShapeDtypeStruct((N, HID), jnp.float32),
            jax.ShapeDtypeStruct((N, 1), jnp.float32),
            jax.ShapeDtypeStruct((N, 1), jnp.float32),
            jax.ShapeDtypeStruct((1, HID), jnp.float32),
        ],
    )(x, wint, b, lg, lb, w1t, a_s, a_d)


def _tc2_body(h_ref, agg_ref, ng_ref, nb_ref, wt_ref, as_ref, ad_ref,
              hn_ref, hw_ref, s1_ref, s2_ref, cb_ref):
    agg = agg_ref[0] + agg_ref[1]
    hn = _ln(h_ref[...] + _elu(agg), ng_ref[...], nb_ref[...])
    hn_ref[...] = hn
    hw = jnp.dot(hn, wt_ref[...], preferred_element_type=jnp.float32)
    hw_ref[...] = hw
    _scores(hw, as_ref[...], ad_ref[...], s1_ref, s2_ref, cb_ref)


def _tc2(h, agg, ng, nb, wt, a_s, a_d):
    return pl.pallas_call(
        _tc2_body,
        out_shape=[
            jax.ShapeDtypeStruct((N, HID), jnp.float32),
            jax.ShapeDtypeStruct((N, HID), jnp.float32),
            jax.ShapeDtypeStruct((N, 1), jnp.float32),
            jax.ShapeDtypeStruct((N, 1), jnp.float32),
            jax.ShapeDtypeStruct((1, HID), jnp.float32),
        ],
    )(h, agg, ng, nb, wt, a_s, a_d)


def _tc3_body(h_ref, agg_ref, ng_ref, nb_ref, ws1t_ref, bs1_ref, ws2t_ref,
              bs2_ref, out_ref):
    agg = agg_ref[0] + agg_ref[1]
    h2 = _ln(h_ref[...] + _elu(agg), ng_ref[...], nb_ref[...])
    g = jnp.mean(h2, axis=0, keepdims=True)
    hid = _gelu(jnp.dot(g, ws1t_ref[...], preferred_element_type=jnp.float32)
                + bs1_ref[...])
    out_ref[...] = (jnp.dot(hid, ws2t_ref[...],
                            preferred_element_type=jnp.float32) + bs2_ref[...])


def _tc3(h, agg, ng, nb, ws1t, bs1, ws2t, bs2):
    return pl.pallas_call(
        _tc3_body,
        out_shape=jax.ShapeDtypeStruct((1, 1), jnp.float32),
    )(h, agg, ng, nb, ws1t, bs1, ws2t, bs2)


def _sc_gat_body(src_hbm, dst_hbm, s1_hbm, s2_hbm, cb_hbm, hw_hbm, out_hbm,
                 src_v, dst_v, gidx_v, sidx_v, s1_v, s2_v, cb_v, a_v,
                 denp_v, comb_v, dsl_v, den_v, rows_v, out_v,
                 parts_sh, den_sh, agg_sh, sem):
    cid = lax.axis_index("c")
    sid = lax.axis_index("s")
    nbase = sid * NPT

    pltpu.sync_copy(src_hbm.at[sid], src_v)
    pltpu.sync_copy(dst_hbm.at[sid], dst_v)

    for j in range(NRC):
        for q in range(8):
            kk = cid * (NV // 2) + j * 8 + q
            gidx_v[j, pl.ds(16 * q, 16)] = src_v[kk, pl.ds(0, 16)]
            sidx_v[j, pl.ds(16 * q, 16)] = dst_v[kk, pl.ds(0, 16)]
    gathers = [
        pltpu.async_copy(hw_hbm.at[gidx_v.at[j]],
                         rows_v.at[pl.ds(128 * j, 128)], sem)
        for j in range(NRC)
    ]

    pltpu.sync_copy(s1_hbm, s1_v)
    pltpu.sync_copy(s2_hbm, s2_v)
    pltpu.sync_copy(cb_hbm, cb_v)

    zf = jnp.zeros((16,), jnp.float32)
    for k in range(N // 16):
        denp_v[pl.ds(16 * k, 16)] = zf
    for r in range(NPT):
        for q in range(HID // 16):
            out_v[r, pl.ds(16 * q, 16)] = zf
    pltpu.sync_copy(out_v, agg_sh.at[pl.ds(nbase, NPT)])

    cvec = cb_v[pl.ds(0, 16)]
    emc = jnp.exp(-cvec)
    for k in range(NV):
        i1 = src_v[k, pl.ds(0, 16)]
        i2 = dst_v[k, pl.ds(0, 16)]
        a = plsc.load_gather(s1_v, [i1]) + plsc.load_gather(s2_v, [i2])
        a = jnp.where(a > 0, a, 0.2 * a)
        ea = jnp.exp(a - cvec)
        a_v[pl.ds(16 * k, 16)] = ea
        plsc.addupdate_scatter(denp_v, [i2], ea - emc)
    pltpu.sync_copy(denp_v, parts_sh.at[pl.ds(sid * N, N)])
    plsc.subcore_barrier()

    for w in range(NS):
        pltpu.sync_copy(parts_sh.at[pl.ds(w * N + nbase, NPT)],
                        comb_v.at[pl.ds(w * NPT, NPT)])
    base = jnp.full((16,), float(E), jnp.float32) * emc
    for q in range(NPT // 16):
        acc = base
        for w in range(NS):
            acc = acc + comb_v[pl.ds(w * NPT + 16 * q, 16)]
        dsl_v[pl.ds(16 * q, 16)] = acc
    pltpu.sync_copy(dsl_v, den_sh.at[pl.ds(nbase, NPT)])
    plsc.subcore_barrier()

    pltpu.sync_copy(den_sh, den_v)
    for k in range(NV):
        i2 = dst_v[k, pl.ds(0, 16)]
        d = plsc.load_gather(den_v, [i2])
        a_v[pl.ds(16 * k, 16)] = a_v[pl.ds(16 * k, 16)] / d

    for g in gathers:
        g.wait()
    ebase = cid * EPC

    @plsc.parallel_loop(0, EPC, unroll=4)
    def _(e):
        cf = plsc.load_gather(a_v, [jnp.full((16,), ebase, jnp.int32)
                                    + jnp.full((16,), e, jnp.int32)])
        for q in range(HID // 16):
            rows_v[e, pl.ds(16 * q, 16)] = rows_v[e, pl.ds(16 * q, 16)] * cf

    for j in range(NRC):
        pltpu.sync_copy(rows_v.at[pl.ds(128 * j, 128)],
                        agg_sh.at[sidx_v.at[j]], add=True)
    plsc.subcore_barrier()

    pltpu.sync_copy(agg_sh.at[pl.ds(nbase, NPT)], out_v)
    pltpu.sync_copy(out_v, out_hbm.at[cid, pl.ds(nbase, NPT)])


_SC_SCRATCH = [
    pltpu.VMEM((NV, 16), jnp.int32),
    pltpu.VMEM((NV, 16), jnp.int32),
    pltpu.VMEM((NRC, 128), jnp.int32),
    pltpu.VMEM((NRC, 128), jnp.int32),
    pltpu.VMEM((N,), jnp.float32),
    pltpu.VMEM((N,), jnp.float32),
    pltpu.VMEM((HID,), jnp.float32),
    pltpu.VMEM((EPT,), jnp.float32),
    pltpu.VMEM((N,), jnp.float32),
    pltpu.VMEM((NS * NPT,), jnp.float32),
    pltpu.VMEM((NPT,), jnp.float32),
    pltpu.VMEM((N,), jnp.float32),
    pltpu.VMEM((EPC, HID), jnp.float32),
    pltpu.VMEM((NPT, HID), jnp.float32),
    pltpu.VMEM_SHARED((NS * N,), jnp.float32),
    pltpu.VMEM_SHARED((N,), jnp.float32),
    pltpu.VMEM_SHARED((N, HID), jnp.float32),
    pltpu.SemaphoreType.DMA,
]


@functools.lru_cache(maxsize=1)
def _build_sc_gat():
    return functools.partial(
        pl.kernel,
        out_type=jax.ShapeDtypeStruct((2, N, HID), jnp.float32),
        mesh=plsc.VectorSubcoreMesh(core_axis_name="c", subcore_axis_name="s"),
        scratch_types=_SC_SCRATCH,
        compiler_params=pltpu.CompilerParams(needs_layout_passes=False),
    )(_sc_gat_body)


def _sc_gat(src3, dst3, s1, s2, cb, hw):
    return _build_sc_gat()(src3, dst3, s1, s2, cb, hw)


def kernel(x, edge_index, edge_attr, W_in, b_in, ln_g, ln_b, emb, W1, att1,
           n1_g, n1_b, W2, att2, n2_g, n2_b, Ws1, bs1, Ws2, bs2):
    src3 = edge_index[0].reshape(NS, NV, 16)
    dst3 = edge_index[1].reshape(NS, NV, 16)
    r1 = lambda v: v.reshape(1, -1)

    h, hw1, s1a, s2a, cb1 = _tc1(x, W_in.T, r1(b_in), r1(ln_g), r1(ln_b),
                                 W1.T, att1[:, :HID], att1[:, HID:])
    agg1 = _sc_gat(src3, dst3, s1a.reshape(N), s2a.reshape(N),
                   cb1.reshape(HID), hw1)
    h1, hw2, s1b, s2b, cb2 = _tc2(h, agg1, r1(n1_g), r1(n1_b), W2.T,
                                  att2[:, :HID], att2[:, HID:])
    agg2 = _sc_gat(src3, dst3, s1b.reshape(N), s2b.reshape(N),
                   cb2.reshape(HID), hw2)
    return _tc3(h1, agg2, r1(n2_g), r1(n2_b), Ws1.T, r1(bs1), Ws2.T,
                bs2.reshape(1, 1))


# speedup vs baseline: 12.7184x; 1.1376x over previous
import functools

import jax
import jax.numpy as jnp
from jax import lax
from jax.experimental import pallas as pl
from jax.experimental.pallas import tpu as pltpu
from jax.experimental.pallas import tpu_sc as plsc

N = 1024
E = 16384
FEAT = 256
HID = 128
NS = 16
NPT = N // NS
EPT = E // 32
NV = EPT // 16
NRC = EPT // 128


def _ln(t, g, b):
    m = jnp.mean(t, axis=-1, keepdims=True)
    v = jnp.mean((t - m) ** 2, axis=-1, keepdims=True)
    return (t - m) * lax.rsqrt(v + 1e-5) * g + b


def _gelu(t):
    return 0.5 * t * (1.0 + lax.erf(t / jnp.sqrt(2.0).astype(jnp.float32)))


def _elu(t):
    return jnp.where(t > 0, t, jnp.exp(t) - 1.0)


def _scores(hw, a_s, a_d, s1_ref, s2_ref, cb_ref):
    s1 = jnp.sum(hw * a_s, axis=-1, keepdims=True)
    s2 = jnp.sum(hw * a_d, axis=-1, keepdims=True)
    s1_ref[...] = s1
    s2_ref[...] = s2
    cb = jnp.maximum(0.0, jnp.max(s1) + jnp.max(s2))
    cb_ref[...] = jnp.zeros((1, HID), jnp.float32) + cb


def _combine(agg_ref, denp_ref, cb_ref):
    p = denp_ref[0] + denp_ref[1]
    emc = jnp.exp(-cb_ref[...][:, :1])
    den = lax.dot_general(p, jnp.ones((NS, 1), jnp.float32),
                          (((0,), (0,)), ((), ())),
                          preferred_element_type=jnp.float32)
    den = den + float(E) * emc
    return (agg_ref[0] + agg_ref[1]) / den


def _tc1_body(x_ref, wint_ref, b_ref, lg_ref, lb_ref, w1t_ref, as_ref, ad_ref,
              h_ref, hw_ref, s1_ref, s2_ref, cb_ref):
    t = jnp.dot(x_ref[...], wint_ref[...], preferred_element_type=jnp.float32)
    h = _gelu(_ln(t + b_ref[...], lg_ref[...], lb_ref[...]))
    h_ref[...] = h
    hw = jnp.dot(h, w1t_ref[...], preferred_element_type=jnp.float32)
    hw_ref[...] = hw
    _scores(hw, as_ref[...], ad_ref[...], s1_ref, s2_ref, cb_ref)


def _tc1(x, wint, b, lg, lb, w1t, a_s, a_d):
    return pl.pallas_call(
        _tc1_body,
        out_shape=[
            jax.ShapeDtypeStruct((N, HID), jnp.float32),
            jax.ShapeDtypeStruct((N, HID), jnp.float32),
            jax.ShapeDtypeStruct((N, 1), jnp.float32),
            jax.ShapeDtypeStruct((N, 1), jnp.float32),
            jax.ShapeDtypeStruct((1, HID), jnp.float32),
        ],
    )(x, wint, b, lg, lb, w1t, a_s, a_d)


def _tc2_body(h_ref, agg_ref, denp_ref, cbp_ref, ng_ref, nb_ref, wt_ref,
              as_ref, ad_ref, hn_ref, hw_ref, s1_ref, s2_ref, cb_ref):
    agg = _combine(agg_ref, denp_ref, cbp_ref)
    hn = _ln(h_ref[...] + _elu(agg), ng_ref[...], nb_ref[...])
    hn_ref[...] = hn
    hw = jnp.dot(hn, wt_ref[...], preferred_element_type=jnp.float32)
    hw_ref[...] = hw
    _scores(hw, as_ref[...], ad_ref[...], s1_ref, s2_ref, cb_ref)


def _tc2(h, agg, denp, cbp, ng, nb, wt, a_s, a_d):
    return pl.pallas_call(
        _tc2_body,
        out_shape=[
            jax.ShapeDtypeStruct((N, HID), jnp.float32),
            jax.ShapeDtypeStruct((N, HID), jnp.float32),
            jax.ShapeDtypeStruct((N, 1), jnp.float32),
            jax.ShapeDtypeStruct((N, 1), jnp.float32),
            jax.ShapeDtypeStruct((1, HID), jnp.float32),
        ],
    )(h, agg, denp, cbp, ng, nb, wt, a_s, a_d)


def _tc3_body(h_ref, agg_ref, denp_ref, cbp_ref, ng_ref, nb_ref, ws1t_ref,
              bs1_ref, ws2t_ref, bs2_ref, out_ref):
    agg = _combine(agg_ref, denp_ref, cbp_ref)
    h2 = _ln(h_ref[...] + _elu(agg), ng_ref[...], nb_ref[...])
    g = jnp.mean(h2, axis=0, keepdims=True)
    hid = _gelu(jnp.dot(g, ws1t_ref[...], preferred_element_type=jnp.float32)
                + bs1_ref[...])
    out_ref[...] = (jnp.dot(hid, ws2t_ref[...],
                            preferred_element_type=jnp.float32) + bs2_ref[...])


def _tc3(h, agg, denp, cbp, ng, nb, ws1t, bs1, ws2t, bs2):
    return pl.pallas_call(
        _tc3_body,
        out_shape=jax.ShapeDtypeStruct((1, 1), jnp.float32),
    )(h, agg, denp, cbp, ng, nb, ws1t, bs1, ws2t, bs2)


def _sc_gat_body(src_hbm, dst_hbm, s1_hbm, s2_hbm, cb_hbm, hw_hbm,
                 agg_hbm, denp_hbm,
                 src_v, dst_v, gidx_v, sidx_v, s1_v, s2_v, cb_v, a_v,
                 denp_v, rows_v, out_v, agg_sh, sem):
    cid = lax.axis_index("c")
    sid = lax.axis_index("s")
    nbase = sid * NPT

    pltpu.sync_copy(src_hbm.at[cid, sid], src_v)
    pltpu.sync_copy(dst_hbm.at[cid, sid], dst_v)
    for j in range(NRC):
        for q in range(8):
            kk = j * 8 + q
            gidx_v[j, pl.ds(16 * q, 16)] = src_v[kk, pl.ds(0, 16)]
            sidx_v[j, pl.ds(16 * q, 16)] = dst_v[kk, pl.ds(0, 16)]
    gathers = [
        pltpu.async_copy(hw_hbm.at[gidx_v.at[j]],
                         rows_v.at[pl.ds(128 * j, 128)], sem)
        for j in range(NRC)
    ]

    pltpu.sync_copy(s1_hbm, s1_v)
    pltpu.sync_copy(s2_hbm, s2_v)
    pltpu.sync_copy(cb_hbm, cb_v)

    zf = jnp.zeros((16,), jnp.float32)
    for k in range(N // 16):
        denp_v[pl.ds(16 * k, 16)] = zf
    for r in range(NPT):
        for q in range(HID // 16):
            out_v[r, pl.ds(16 * q, 16)] = zf
    pltpu.sync_copy(out_v, agg_sh.at[pl.ds(nbase, NPT)])
    plsc.subcore_barrier()

    cvec = cb_v[pl.ds(0, 16)]
    emc = jnp.exp(-cvec)
    for k in range(NV):
        i1 = src_v[k, pl.ds(0, 16)]
        i2 = dst_v[k, pl.ds(0, 16)]
        a = plsc.load_gather(s1_v, [i1]) + plsc.load_gather(s2_v, [i2])
        a = jnp.where(a > 0, a, 0.2 * a)
        ea = jnp.exp(a - cvec)
        a_v[pl.ds(16 * k, 16)] = ea
        plsc.addupdate_scatter(denp_v, [i2], ea - emc)
    pltpu.sync_copy(denp_v, denp_hbm.at[cid, sid])

    for g in gathers:
        g.wait()

    @plsc.parallel_loop(0, EPT, unroll=4)
    def _(e):
        cf = plsc.load_gather(a_v, [jnp.full((16,), e, jnp.int32)])
        for q in range(HID // 16):
            rows_v[e, pl.ds(16 * q, 16)] = rows_v[e, pl.ds(16 * q, 16)] * cf

    for j in range(NRC):
        pltpu.sync_copy(rows_v.at[pl.ds(128 * j, 128)],
                        agg_sh.at[sidx_v.at[j]], add=True)
    plsc.subcore_barrier()

    pltpu.sync_copy(agg_sh.at[pl.ds(nbase, NPT)], out_v)
    pltpu.sync_copy(out_v, agg_hbm.at[cid, pl.ds(nbase, NPT)])


_SC_SCRATCH = [
    pltpu.VMEM((NV, 16), jnp.int32),
    pltpu.VMEM((NV, 16), jnp.int32),
    pltpu.VMEM((NRC, 128), jnp.int32),
    pltpu.VMEM((NRC, 128), jnp.int32),
    pltpu.VMEM((N,), jnp.float32),
    pltpu.VMEM((N,), jnp.float32),
    pltpu.VMEM((HID,), jnp.float32),
    pltpu.VMEM((EPT,), jnp.float32),
    pltpu.VMEM((N,), jnp.float32),
    pltpu.VMEM((EPT, HID), jnp.float32),
    pltpu.VMEM((NPT, HID), jnp.float32),
    pltpu.VMEM_SHARED((N, HID), jnp.float32),
    pltpu.SemaphoreType.DMA,
]


@functools.lru_cache(maxsize=1)
def _build_sc_gat():
    return functools.partial(
        pl.kernel,
        out_type=[
            jax.ShapeDtypeStruct((2, N, HID), jnp.float32),
            jax.ShapeDtypeStruct((2, NS, N), jnp.float32),
        ],
        mesh=plsc.VectorSubcoreMesh(core_axis_name="c", subcore_axis_name="s"),
        scratch_types=_SC_SCRATCH,
        compiler_params=pltpu.CompilerParams(needs_layout_passes=False),
    )(_sc_gat_body)


def _sc_gat(src4, dst4, s1, s2, cb, hw):
    return _build_sc_gat()(src4, dst4, s1, s2, cb, hw)


def kernel(x, edge_index, edge_attr, W_in, b_in, ln_g, ln_b, emb, W1, att1,
           n1_g, n1_b, W2, att2, n2_g, n2_b, Ws1, bs1, Ws2, bs2):
    src4 = edge_index[0].reshape(2, NS, NV, 16)
    dst4 = edge_index[1].reshape(2, NS, NV, 16)
    r1 = lambda v: v.reshape(1, -1)

    h, hw1, s1a, s2a, cb1 = _tc1(x, W_in.T, r1(b_in), r1(ln_g), r1(ln_b),
                                 W1.T, att1[:, :HID], att1[:, HID:])
    agg1, dp1 = _sc_gat(src4, dst4, s1a.reshape(N), s2a.reshape(N),
                        cb1.reshape(HID), hw1)
    h1, hw2, s1b, s2b, cb2 = _tc2(h, agg1, dp1, cb1, r1(n1_g), r1(n1_b),
                                  W2.T, att2[:, :HID], att2[:, HID:])
    agg2, dp2 = _sc_gat(src4, dst4, s1b.reshape(N), s2b.reshape(N),
                        cb2.reshape(HID), hw2)
    return _tc3(h1, agg2, dp2, cb2, r1(n2_g), r1(n2_b), Ws1.T, r1(bs1),
                Ws2.T, bs2.reshape(1, 1))
